# baseline (speedup 1.0000x reference)
import functools

import jax
import jax.numpy as jnp
from jax import lax
from jax.experimental import pallas as pl
from jax.experimental.pallas import tpu as pltpu
from jax.experimental.pallas import tpu_sc as plsc

B = 4096
V = 100000
D = 64
L = 20
NC = 2
NS = 16
NW = NC * NS
BPW = B // NW
LANES = 16


def _sc_pooled_lookup(uidx2, midx3, user_table, movie_table):
  mesh = plsc.VectorSubcoreMesh(
      core_axis_name="c", subcore_axis_name="s",
      num_cores=NC, num_subcores=NS)

  @functools.partial(
      pl.kernel,
      out_type=(jax.ShapeDtypeStruct((B, D), jnp.float32),
                jax.ShapeDtypeStruct((B, D), jnp.float32)),
      mesh=mesh,
      compiler_params=pltpu.CompilerParams(
          use_tc_tiling_on_sc=False, needs_layout_passes=False),
      scratch_types=[
          pltpu.VMEM((BPW,), jnp.int32),
          pltpu.VMEM((BPW * L,), jnp.int32),
          pltpu.VMEM((L, BPW), jnp.int32),
          pltpu.VMEM((BPW, D), jnp.float32),
          pltpu.VMEM((BPW, D), jnp.float32),
          pltpu.VMEM((BPW, D), jnp.float32),
          pltpu.VMEM((BPW, D), jnp.float32),
          pltpu.SemaphoreType.DMA,
          pltpu.SemaphoreType.DMA,
          pltpu.SemaphoreType.DMA,
      ],
  )
  def k(uidx_hbm, midx_hbm, ut_hbm, mt_hbm, ue_out, me_out,
        uidx_v, midx_f, midx_v, urows, mbuf0, mbuf1, pooled,
        usem, msem0, msem1):
    wid = lax.axis_index("s") * NC + lax.axis_index("c")
    base = wid * BPW

    pltpu.sync_copy(uidx_hbm.at[wid], uidx_v)
    pltpu.sync_copy(midx_hbm.at[wid], midx_f)
    lanes = lax.iota(jnp.int32, LANES) * L
    for l in range(L):
      for j0 in range(BPW // LANES):
        vals = plsc.load_gather(midx_f, [lanes + (j0 * LANES * L + l)])
        midx_v[l, pl.ds(j0 * LANES, LANES)] = vals

    ucopy = pltpu.async_copy(ut_hbm.at[uidx_v], urows, usem)

    bufs = (mbuf0, mbuf1)
    sems = (msem0, msem1)
    copies = [None] * L
    for l in range(2):
      copies[l] = pltpu.async_copy(
          mt_hbm.at[midx_v.at[l]], bufs[l % 2], sems[l % 2])

    for l in range(L):
      copies[l].wait()
      buf = bufs[l % 2]
      if l == 0:
        def init_body(r, _, buf=buf):
          for g in range(D // LANES):
            pooled[r, pl.ds(g * LANES, LANES)] = buf[r, pl.ds(g * LANES, LANES)]
          return 0
        lax.fori_loop(0, BPW, init_body, 0)
      else:
        def acc_body(r, _, buf=buf):
          for g in range(D // LANES):
            plsc.addupdate(pooled.at[r, pl.ds(g * LANES, LANES)],
                           buf[r, pl.ds(g * LANES, LANES)])
          return 0
        lax.fori_loop(0, BPW, acc_body, 0)
      if l + 2 < L:
        copies[l + 2] = pltpu.async_copy(
            mt_hbm.at[midx_v.at[l + 2]], bufs[l % 2], sems[l % 2])

    pltpu.sync_copy(pooled, me_out.at[pl.ds(base, BPW)])
    ucopy.wait()
    pltpu.sync_copy(urows, ue_out.at[pl.ds(base, BPW)])

  return k(uidx2, midx3, user_table, movie_table)


def _mlp(ue, me, W1, b1, W2, b2, W3, b3):
  BM = 512

  def body(ue_ref, me_ref, w1_ref, b1_ref, w2_ref, b2_ref, w3_ref, b3_ref,
           o_ref):
    h = jnp.dot(ue_ref[...], w1_ref[:D], preferred_element_type=jnp.float32)
    h = h + jnp.dot(me_ref[...], w1_ref[D:], preferred_element_type=jnp.float32)
    h = jax.nn.relu(h + b1_ref[...])
    h = jax.nn.relu(
        jnp.dot(h, w2_ref[...], preferred_element_type=jnp.float32)
        + b2_ref[...])
    o_ref[...] = (jnp.dot(h, w3_ref[...], preferred_element_type=jnp.float32)
                  + b3_ref[...])

  grid = (B // BM,)
  return pl.pallas_call(
      body,
      grid=grid,
      in_specs=[
          pl.BlockSpec((BM, D), lambda i: (i, 0)),
          pl.BlockSpec((BM, D), lambda i: (i, 0)),
          pl.BlockSpec((2 * D, 256), lambda i: (0, 0)),
          pl.BlockSpec((1, 256), lambda i: (0, 0)),
          pl.BlockSpec((256, 128), lambda i: (0, 0)),
          pl.BlockSpec((1, 128), lambda i: (0, 0)),
          pl.BlockSpec((128, 1), lambda i: (0, 0)),
          pl.BlockSpec((1, 1), lambda i: (0, 0)),
      ],
      out_specs=pl.BlockSpec((BM, 1), lambda i: (i, 0)),
      out_shape=jax.ShapeDtypeStruct((B, 1), jnp.float32),
  )(ue, me, W1, b1.reshape(1, 256), W2, b2.reshape(1, 128), W3,
    b3.reshape(1, 1))


def kernel(user_indices, movie_indices, user_table, movie_table,
           W1, b1, W2, b2, W3, b3):
  uidx2 = user_indices.astype(jnp.int32).reshape(NW, BPW)
  midx2 = movie_indices.astype(jnp.int32).reshape(NW, BPW * L)
  ue, me = _sc_pooled_lookup(uidx2, midx2, user_table, movie_table)
  pred = _mlp(ue, me, W1, b1, W2, b2, W3, b3)
  return pred.squeeze(-1)


# speedup vs baseline: 1.0192x; 1.0192x over previous
import functools

import jax
import jax.numpy as jnp
from jax import lax
from jax.experimental import pallas as pl
from jax.experimental.pallas import tpu as pltpu
from jax.experimental.pallas import tpu_sc as plsc

B = 4096
V = 100000
D = 64
L = 20
NC = 2
NS = 16
NW = NC * NS
BPW = B // NW
LANES = 16
NBUF = 4


def _sc_pooled_lookup(user_indices, midx_t, user_table, movie_table):
  mesh = plsc.VectorSubcoreMesh(
      core_axis_name="c", subcore_axis_name="s",
      num_cores=NC, num_subcores=NS)

  @functools.partial(
      pl.kernel,
      out_type=(jax.ShapeDtypeStruct((B, D), jnp.float32),
                jax.ShapeDtypeStruct((B, D), jnp.float32)),
      mesh=mesh,
      compiler_params=pltpu.CompilerParams(use_tc_tiling_on_sc=False),
      scratch_types=[
          pltpu.VMEM((BPW,), jnp.int32),
          pltpu.VMEM((L, BPW), jnp.int32),
          pltpu.VMEM((BPW, D), jnp.float32),
          pltpu.VMEM((NBUF, BPW, D), jnp.float32),
          pltpu.VMEM((BPW, D), jnp.float32),
          pltpu.SemaphoreType.DMA,
          [pltpu.SemaphoreType.DMA] * NBUF,
      ],
  )
  def k(uidx_hbm, midx_hbm, ut_hbm, mt_hbm, ue_out, me_out,
        uidx_v, midx_v, urows, mring, pooled, usem, msems):
    wid = lax.axis_index("s") * NC + lax.axis_index("c")
    base = wid * BPW

    pltpu.sync_copy(uidx_hbm.at[pl.ds(base, BPW)], uidx_v)
    pltpu.sync_copy(midx_hbm.at[:, pl.ds(base, BPW)], midx_v)

    ucopy = pltpu.async_copy(ut_hbm.at[uidx_v], urows, usem)

    copies = [None] * L
    for l in range(NBUF):
      copies[l] = pltpu.async_copy(
          mt_hbm.at[midx_v.at[l]], mring.at[l % NBUF], msems[l % NBUF])

    zeros = jnp.zeros((LANES,), jnp.float32)

    @plsc.parallel_loop(0, BPW, unroll=4)
    def _(r):
      for g in range(D // LANES):
        pooled[r, pl.ds(g * LANES, LANES)] = zeros

    for l in range(L):
      copies[l].wait()
      buf = mring.at[l % NBUF]

      @plsc.parallel_loop(0, BPW, unroll=4)
      def _(r, buf=buf):
        for g in range(D // LANES):
          plsc.addupdate(pooled.at[r, pl.ds(g * LANES, LANES)],
                         buf[r, pl.ds(g * LANES, LANES)])

      if l + NBUF < L:
        copies[l + NBUF] = pltpu.async_copy(
            mt_hbm.at[midx_v.at[l + NBUF]], mring.at[l % NBUF],
            msems[l % NBUF])

    pltpu.sync_copy(pooled, me_out.at[pl.ds(base, BPW)])
    ucopy.wait()
    pltpu.sync_copy(urows, ue_out.at[pl.ds(base, BPW)])

  return k(user_indices, midx_t, user_table, movie_table)


def _mlp(ue, me, W1, b1, W2, b2, W3, b3):
  BM = 512

  def body(ue_ref, me_ref, w1_ref, b1_ref, w2_ref, b2_ref, w3_ref, b3_ref,
           o_ref):
    h = jnp.dot(ue_ref[...], w1_ref[:D], preferred_element_type=jnp.float32)
    h = h + jnp.dot(me_ref[...], w1_ref[D:], preferred_element_type=jnp.float32)
    h = jax.nn.relu(h + b1_ref[...])
    h = jax.nn.relu(
        jnp.dot(h, w2_ref[...], preferred_element_type=jnp.float32)
        + b2_ref[...])
    o_ref[...] = (jnp.dot(h, w3_ref[...], preferred_element_type=jnp.float32)
                  + b3_ref[...])

  grid = (B // BM,)
  return pl.pallas_call(
      body,
      grid=grid,
      in_specs=[
          pl.BlockSpec((BM, D), lambda i: (i, 0)),
          pl.BlockSpec((BM, D), lambda i: (i, 0)),
          pl.BlockSpec((2 * D, 256), lambda i: (0, 0)),
          pl.BlockSpec((1, 256), lambda i: (0, 0)),
          pl.BlockSpec((256, 128), lambda i: (0, 0)),
          pl.BlockSpec((1, 128), lambda i: (0, 0)),
          pl.BlockSpec((128, 1), lambda i: (0, 0)),
          pl.BlockSpec((1, 1), lambda i: (0, 0)),
      ],
      out_specs=pl.BlockSpec((BM, 1), lambda i: (i, 0)),
      out_shape=jax.ShapeDtypeStruct((B, 1), jnp.float32),
  )(ue, me, W1, b1.reshape(1, 256), W2, b2.reshape(1, 128), W3,
    b3.reshape(1, 1))


def kernel(user_indices, movie_indices, user_table, movie_table,
           W1, b1, W2, b2, W3, b3):
  midx_t = movie_indices.astype(jnp.int32).T
  ue, me = _sc_pooled_lookup(
      user_indices.astype(jnp.int32), midx_t, user_table, movie_table)
  pred = _mlp(ue, me, W1, b1, W2, b2, W3, b3)
  return pred.squeeze(-1)
